# baseline (speedup 1.0000x reference)
import jax
import jax.numpy as jnp
from jax.experimental import pallas as pl
from jax.experimental.pallas import tpu as pltpu

_N = 16384
_K = 64 * 50
_A = 21
_B = 1024
_R = 128


def _fused_kernel(x_ref, w_ref, u_ref, o_ref):
    i = pl.program_id(0)
    z = jnp.dot(x_ref[...], w_ref[...], preferred_element_type=jnp.float32)
    probs = jax.nn.softmax(z, axis=-1)
    logits = jnp.log(probs + 1e-08)
    gumbel = -jnp.log(-jnp.log(u_ref[...]))
    y = jax.nn.softmax(logits + gumbel, axis=-1)
    idx = jnp.argmax(y, axis=-1)
    acts = idx.astype(jnp.float32) * jnp.float32(0.05)
    rows = _B // (_N // _R)
    o_ref[pl.ds(i * rows, rows), :] = acts.reshape(rows, _N // _R)

    @pl.when(i == (_N // _B) - 1)
    def _normalize():
        a = o_ref[...]
        r = jax.lax.broadcasted_iota(jnp.int32, (_R, _N // _R), 0)
        c = jax.lax.broadcasted_iota(jnp.int32, (_R, _N // _R), 1)
        is0 = (r == 0) & (c == 0)
        s = jnp.sum(jnp.where(is0, 0.0, a))
        scale = jnp.where(s > 1.0, 1.0 / s, 1.0)
        scaled = a * scale
        s2 = jnp.sum(jnp.where(is0, 0.0, scaled))
        o_ref[...] = jnp.where(is0, 1.0 - s2, scaled)


def kernel(x, W):
    feats = x.reshape(_N, _K)
    u = jax.random.uniform(jax.random.key(1), (_N, _A), minval=1e-10, maxval=1.0)
    out = pl.pallas_call(
        _fused_kernel,
        grid=(_N // _B,),
        in_specs=[
            pl.BlockSpec((_B, _K), lambda i: (i, 0)),
            pl.BlockSpec((_K, _A), lambda i: (0, 0)),
            pl.BlockSpec((_B, _A), lambda i: (i, 0)),
        ],
        out_specs=pl.BlockSpec((_R, _N // _R), lambda i: (0, 0)),
        out_shape=jax.ShapeDtypeStruct((_R, _N // _R), jnp.float32),
        compiler_params=pltpu.CompilerParams(
            dimension_semantics=("arbitrary",),
        ),
    )(feats, W, u)
    return out.reshape(_N)
